# Initial kernel scaffold; baseline (speedup 1.0000x reference)
import jax
import jax.numpy as jnp
from jax.experimental import pallas as pl


def kernel(x, edge_attr, edge_index, W_edge, b_edge, W_conv, ln_gamma, ln_beta):
    raise NotImplementedError("write your pallas kernel here")


# speedup vs baseline: 1.4534x; 1.4534x over previous
import functools

import jax
import jax.numpy as jnp
from jax import lax
from jax.experimental import pallas as pl
from jax.experimental.pallas import tpu as pltpu
from jax.experimental.pallas import tpu_sc as plsc

N = 10000
E = 160000
BC = 8
H = 60
EC = 4
HP = 64
DP = BC * HP
NVR = DP // 16
D = BC * H

NC = 2
NS = 16
NW = NC * NS

BK = 40
EW = E // NW
NBW = EW // BK
ES = E // NS
BKS = 80
NBS2 = ES // BKS
NCHUNK = 8
NP = 10240

_RN = float(1.0 / D)
_EPS = 1e-5


def _uv_body(x_ref, wu_ref, wv_ref, u_ref, v_ref):
    xb = x_ref[...]
    u_ref[...] = jnp.dot(xb, wu_ref[...], preferred_element_type=jnp.float32)
    v_ref[...] = jnp.dot(xb, wv_ref[...], preferred_element_type=jnp.float32)


def _ce_body(ea_ref, m_ref, o_ref):
    o_ref[...] = jnp.dot(ea_ref[...], m_ref[...],
                         preferred_element_type=jnp.float32)


def _lane_splat(vec, ch):
    idx = jnp.full((16, 1), ch, dtype=jnp.int32)
    dn = lax.GatherDimensionNumbers(
        offset_dims=(), collapsed_slice_dims=(0,), start_index_map=(0,))
    return lax.gather(vec, idx, dn, slice_sizes=(1,),
                      mode=lax.GatherScatterMode.PROMISE_IN_BOUNDS)


def _newton_rsqrt(v):
    i = plsc.bitcast(v, jnp.int32)
    i = jnp.int32(0x5F3759DF) - (i >> 1)
    r = plsc.bitcast(i, jnp.float32)
    half = jnp.float32(0.5)
    three_half = jnp.float32(1.5)
    for _ in range(3):
        r = r * (three_half - half * v * r * r)
    return r


def _gather_ln_kernel(u_hbm, v_hbm, c_hbm, src_hbm, dst_hbm, g_hbm, b_hbm,
                      msg_hbm, dst_v, src_v, c_v, u_v, v_v, y_v, g_v, b_v,
                      sem):
    wid = lax.axis_index("s") * NC + lax.axis_index("c")
    base = wid * EW
    pltpu.sync_copy(g_hbm, g_v)
    pltpu.sync_copy(b_hbm, b_v)

    lane = lax.iota(jnp.int32, 16)
    pad_mask = lane < jnp.int32(H - 48)

    def block_body(nb, carry):
        e0 = base + nb * BK
        pltpu.sync_copy(dst_hbm.at[pl.ds(e0, BK)], dst_v)
        pltpu.sync_copy(src_hbm.at[pl.ds(e0, BK)], src_v)
        pltpu.sync_copy(c_hbm.at[pl.ds(16 * e0, 16 * BK)], c_v)
        cp_u = pltpu.async_copy(u_hbm.at[dst_v], u_v, sem)
        cp_v = pltpu.async_copy(v_hbm.at[src_v], v_v, sem)
        cp_u.wait()
        cp_v.wait()

        def edge_body(e, carry2):
            c16 = c_v[pl.ds(16 * e, 16)]
            csp = [_lane_splat(c16, ch) for ch in range(BC)]
            zsum = jnp.zeros((16,), jnp.float32)
            zsq = jnp.zeros((16,), jnp.float32)
            trs = []
            for r in range(NVR):
                t = u_v[e, pl.ds(16 * r, 16)] + v_v[e, pl.ds(16 * r, 16)]
                t = t + csp[r // 4]
                if r % 4 == 3:
                    t = jnp.where(pad_mask, t, jnp.float32(0.0))
                trs.append(t)
                zsum = zsum + t
                zsq = zsq + t * t
            mean_v = _lane_splat(plsc.cumsum(zsum), 15) * jnp.float32(_RN)
            ssm_v = _lane_splat(plsc.cumsum(zsq), 15) * jnp.float32(_RN)
            var_v = ssm_v - mean_v * mean_v
            rstd_v = _newton_rsqrt(var_v + jnp.float32(_EPS))
            for r in range(NVR):
                y = (trs[r] - mean_v) * rstd_v * g_v[pl.ds(16 * r, 16)]
                y = y + b_v[pl.ds(16 * r, 16)]
                y_v[r // 4, e, pl.ds(16 * (r % 4), 16)] = \
                    jnp.maximum(y, jnp.float32(0.0))
            return carry2

        lax.fori_loop(0, BK, edge_body, 0)
        for ch in range(NCHUNK):
            pltpu.sync_copy(y_v.at[ch], msg_hbm.at[ch, pl.ds(e0, BK)])
        return carry

    lax.fori_loop(0, NBW, block_body, 0)


def _scatter_kernel(msg_hbm, dsti_hbm, out_hbm, idx_v, m_v, zf_v, acc_s):
    core = lax.axis_index("c")
    s = lax.axis_index("s")
    zv = jnp.zeros((16,), jnp.float32)

    for k in range(NCHUNK // NC):
        chunk = core * (NCHUNK // NC) + k

        def zb(i, c):
            for r in range(4):
                zf_v[i, pl.ds(16 * r, 16)] = zv
            return c

        lax.fori_loop(0, 160, zb, 0)
        for j in range(4):
            pltpu.sync_copy(zf_v, acc_s.at[pl.ds(s * 640 + j * 160, 160)])
        plsc.subcore_barrier()

        def block_body(nb, carry):
            e0 = chunk * E + s * ES + nb * BKS
            pltpu.sync_copy(dsti_hbm.at[pl.ds(s * ES + nb * BKS, BKS)],
                            idx_v)
            pltpu.sync_copy(msg_hbm.at[pl.ds(e0, BKS)], m_v)
            pltpu.sync_copy(m_v, acc_s.at[idx_v], add=True)
            return carry

        lax.fori_loop(0, NBS2, block_body, 0)
        plsc.subcore_barrier()
        for j in range(4):
            rows = pl.ds(s * 640 + j * 160, 160)
            pltpu.sync_copy(acc_s.at[rows], zf_v)
            pltpu.sync_copy(
                zf_v, out_hbm.at[pl.ds(chunk * NP + s * 640 + j * 160, 160)])
        plsc.subcore_barrier()


def kernel(x, edge_attr, edge_index, W_edge, b_edge, W_conv, ln_gamma, ln_beta):
    f32 = jnp.float32
    x = x.astype(f32)

    A = W_conv[:, :BC]
    B_ = W_conv[:, BC:2 * BC]
    C_ = W_conv[:, 2 * BC:]
    eye = jnp.eye(H, HP, dtype=f32)
    W_U = jnp.einsum("oi,hg->ihog", A, eye).reshape(D, DP)
    W_V = jnp.einsum("oi,hg->ihog", B_, eye).reshape(D, DP)
    Mc = C_ @ W_edge
    bc = C_ @ b_edge
    Mfull = jnp.zeros((8, 16), f32)
    Mfull = Mfull.at[:EC, :BC].set(Mc.T)
    Mfull = Mfull.at[EC, :BC].set(bc)
    ea1 = jnp.concatenate(
        [edge_attr.astype(f32),
         jnp.ones((E, 1), f32),
         jnp.zeros((E, 3), f32)], axis=1)
    gp = jnp.pad(ln_gamma, ((0, 0), (0, HP - H))).reshape(DP)
    bp = jnp.pad(ln_beta, ((0, 0), (0, HP - H))).reshape(DP)

    RB = 400
    U, V = pl.pallas_call(
        _uv_body,
        grid=(N // RB,),
        in_specs=[
            pl.BlockSpec((RB, D), lambda i: (i, 0)),
            pl.BlockSpec((D, DP), lambda i: (0, 0)),
            pl.BlockSpec((D, DP), lambda i: (0, 0)),
        ],
        out_specs=[
            pl.BlockSpec((RB, DP), lambda i: (i, 0)),
            pl.BlockSpec((RB, DP), lambda i: (i, 0)),
        ],
        out_shape=[
            jax.ShapeDtypeStruct((N, DP), f32),
            jax.ShapeDtypeStruct((N, DP), f32),
        ],
    )(x, W_U, W_V)

    EB = 8000
    Ce = pl.pallas_call(
        _ce_body,
        grid=(E // EB,),
        in_specs=[
            pl.BlockSpec((EB, 8), lambda i: (i, 0)),
            pl.BlockSpec((8, 16), lambda i: (0, 0)),
        ],
        out_specs=pl.BlockSpec((EB, 16), lambda i: (i, 0)),
        out_shape=jax.ShapeDtypeStruct((E, 16), f32),
    )(ea1, Mfull)

    src = edge_index[0].astype(jnp.int32)
    dst = edge_index[1].astype(jnp.int32)

    mesh = plsc.VectorSubcoreMesh(core_axis_name="c", subcore_axis_name="s")

    msg = pl.kernel(
        _gather_ln_kernel,
        out_type=jax.ShapeDtypeStruct((NCHUNK, E, HP), f32),
        mesh=mesh,
        scratch_types=[
            pltpu.VMEM((BK,), jnp.int32),
            pltpu.VMEM((BK,), jnp.int32),
            pltpu.VMEM((BK * 16,), f32),
            pltpu.VMEM((BK, DP), f32),
            pltpu.VMEM((BK, DP), f32),
            pltpu.VMEM((NCHUNK, BK, HP), f32),
            pltpu.VMEM((DP,), f32),
            pltpu.VMEM((DP,), f32),
            pltpu.SemaphoreType.DMA,
        ],
        compiler_params=pltpu.CompilerParams(needs_layout_passes=False),
    )(U, V, Ce.reshape(E * 16), src, dst, gp, bp)

    msg2 = msg.reshape(NCHUNK * E, HP)
    outp = pl.kernel(
        _scatter_kernel,
        out_type=jax.ShapeDtypeStruct((NCHUNK * NP, 64), f32),
        mesh=mesh,
        scratch_types=[
            pltpu.VMEM((BKS,), jnp.int32),
            pltpu.VMEM((BKS, 64), f32),
            pltpu.VMEM((160, 64), f32),
            pltpu.VMEM_SHARED((NP, 64), f32),
        ],
        compiler_params=pltpu.CompilerParams(needs_layout_passes=False),
    )(msg2, dst)

    out = outp.reshape(NCHUNK, NP, HP)[:, :N].transpose(1, 0, 2)[:, :, :H]
    return out.reshape(N, D)
